# baseline (speedup 1.0000x reference)
import jax
import jax.numpy as jnp
from jax.experimental import pallas as pl

_DIM = 768
_HEADS = 12
_HD = _DIM // _HEADS
_MLPH = 4 * _DIM
_E = 8
_HID = 192
_SCALE = _HD ** -0.5
_ABT = 2
_BT = 4
_CHUNKS = 4
_BF = jnp.bfloat16


def _gelu(x):
    return x * 0.5 * (1.0 + jax.lax.erf(x * (2.0 ** -0.5)))


def _ln(x, g, b):
    m = jnp.mean(x, axis=-1, keepdims=True)
    v = jnp.mean((x - m) ** 2, axis=-1, keepdims=True)
    return (x - m) * jax.lax.rsqrt(v + 1e-5) * g + b


def _dot(a, b):
    return jnp.dot(a, b, preferred_element_type=jnp.float32)


def _block_kernel(x_ref, wq_ref, wkv_ref, wproj_ref, bproj_ref, g1_ref,
                  b1_ref, w1_ref, mb1_ref, w2_ref, mb2_ref, g2_ref, b2_ref,
                  gw_ref, eps_ref, o_ref, es_ref):
    x1s = []
    for i in range(_ABT):
        x = x_ref[i]
        xn = _ln(x, g1_ref[...], b1_ref[...]).astype(_BF)
        qs = _dot(xn, wq_ref[...]).astype(_BF)
        kv = _dot(xn, wkv_ref[...]).astype(_BF)
        outs = []
        for h in range(_HEADS):
            q = qs[:, h * _HD:(h + 1) * _HD]
            k = kv[:, h * _HD:(h + 1) * _HD]
            v = kv[:, _DIM + h * _HD:_DIM + (h + 1) * _HD]
            s = jax.lax.dot_general(q, k, (((1,), (1,)), ((), ())),
                                    preferred_element_type=jnp.float32)
            p = jax.nn.softmax(s * _SCALE, axis=-1)
            outs.append(_dot(p.astype(_BF), v))
        ao = jnp.concatenate(outs, axis=1)
        x1s.append(x + _dot(ao.astype(_BF), wproj_ref[...]) + bproj_ref[...])
    n = x_ref.shape[1]
    x1 = jnp.concatenate(x1s, axis=0)
    xn2 = _ln(x1, g2_ref[...], b2_ref[...])
    acc = x1 + mb2_ref[...]
    ck = _MLPH // _CHUNKS
    for c in range(_CHUNKS):
        h = _gelu(_dot(xn2, w1_ref[:, c * ck:(c + 1) * ck])
                  + mb1_ref[0, c * ck:(c + 1) * ck])
        acc = acc + _dot(h, w2_ref[c * ck:(c + 1) * ck, :])
    o_ref[...] = acc.reshape(_ABT, n, _DIM)
    t = _dot(acc, gw_ref[...])
    clean = t[:, :_E]
    std = jax.nn.softplus(t[:, _E:]) + 0.01
    logits = clean + eps_ref[...].reshape(_ABT * n, _E) * std
    es_ref[0] = logits.reshape(_ABT, n, _E).sum(axis=1)


def _expert_kernel(x_ref, es_ref, w1c_ref, b1c_ref, w2c_ref, b2e_ref, o_ref):
    s = es_ref[...]
    t1 = jnp.max(s, axis=1, keepdims=True)
    ii = jax.lax.broadcasted_iota(jnp.int32, s.shape, 1)
    first = jnp.min(jnp.where(s == t1, ii, _E), axis=1, keepdims=True)
    t2 = jnp.max(jnp.where(ii == first, -jnp.inf, s), axis=1, keepdims=True)
    m2 = jnp.max(t1)
    m1 = jnp.min(t2)
    row0 = jax.lax.broadcasted_iota(jnp.int32, t1.shape, 0) == 0
    t1_0 = jnp.sum(jnp.where(row0, t1, 0.0))
    t2_0 = jnp.sum(jnp.where(row0, t2, 0.0))
    a = (t1_0 - m1) / (m2 - m1)
    b = (t2_0 - m1) / (m2 - m1)
    ea, eb = jnp.exp(a), jnp.exp(b)
    g0 = ea / (ea + eb)
    g1 = eb / (ea + eb)

    bt, n, _ = x_ref.shape
    x2 = x_ref[...].reshape(bt * n, _DIM)
    h = _gelu(_dot(x2.astype(_BF), w1c_ref[...]) + b1c_ref[...])
    gi = jax.lax.broadcasted_iota(jnp.int32, (1, 2 * _HID), 1)
    hs = h * jnp.where(gi < _HID, g0, g1)
    out = (x2 + _dot(hs.astype(_BF), w2c_ref[...])
           + g0 * b2e_ref[0:1, :] + g1 * b2e_ref[1:2, :])
    o_ref[...] = out.reshape(bt, n, _DIM)


def kernel(x, H, W, pseudo_domain_label, norm1_g, norm1_b, norm2_g, norm2_b,
           wq, wkv, wproj, bproj, mlp_w1, mlp_b1, mlp_w2, mlp_b2,
           gate_w, exp_w1, exp_b1, exp_w2, exp_b2):
    B, N, _ = x.shape
    f32 = jnp.float32
    row2 = lambda a: a.reshape(1, -1)

    eps = jax.random.normal(jax.random.key(42), (B, N, _E), f32)
    nstep = B // _ABT
    cw = lambda shp: pl.BlockSpec(shp, lambda r: tuple(0 for _ in shp))
    x2, es3 = pl.pallas_call(
        _block_kernel,
        grid=(nstep,),
        in_specs=[
            pl.BlockSpec((_ABT, N, _DIM), lambda r: (r, 0, 0)),
            cw((_DIM, _DIM)),
            cw((_DIM, 2 * _DIM)),
            cw((_DIM, _DIM)),
            cw((1, _DIM)),
            cw((1, _DIM)),
            cw((1, _DIM)),
            cw((_DIM, _MLPH)),
            cw((1, _MLPH)),
            cw((_MLPH, _DIM)),
            cw((1, _DIM)),
            cw((1, _DIM)),
            cw((1, _DIM)),
            cw((_DIM, 2 * _E)),
            pl.BlockSpec((_ABT, N, _E), lambda r: (r, 0, 0)),
        ],
        out_specs=[
            pl.BlockSpec((_ABT, N, _DIM), lambda r: (r, 0, 0)),
            pl.BlockSpec((1, _ABT, _E), lambda r: (r, 0, 0)),
        ],
        out_shape=[
            jax.ShapeDtypeStruct((B, N, _DIM), f32),
            jax.ShapeDtypeStruct((nstep, _ABT, _E), f32),
        ],
    )(x, wq.astype(_BF), wkv.astype(_BF), wproj.astype(_BF), row2(bproj),
      row2(norm1_g), row2(norm1_b),
      mlp_w1, row2(mlp_b1), mlp_w2, row2(mlp_b2),
      row2(norm2_g), row2(norm2_b), gate_w, eps)
    exp_sums = es3.reshape(B, _E)

    w1c = jnp.concatenate([exp_w1[0], exp_w1[1]], axis=1).astype(_BF)
    b1c = jnp.concatenate([exp_b1[0], exp_b1[1]]).reshape(1, 2 * _HID)
    w2c = jnp.concatenate([exp_w2[0], exp_w2[1]], axis=0).astype(_BF)
    b2e = exp_b2[:2]
    ntile = B // _BT
    x3 = pl.pallas_call(
        _expert_kernel,
        grid=(ntile,),
        in_specs=[
            pl.BlockSpec((_BT, N, _DIM), lambda r: (r, 0, 0)),
            pl.BlockSpec((B, _E), lambda r: (0, 0)),
            pl.BlockSpec((_DIM, 2 * _HID), lambda r: (0, 0)),
            pl.BlockSpec((1, 2 * _HID), lambda r: (0, 0)),
            pl.BlockSpec((2 * _HID, _DIM), lambda r: (0, 0)),
            pl.BlockSpec((2, _DIM), lambda r: (0, 0)),
        ],
        out_specs=pl.BlockSpec((_BT, N, _DIM), lambda r: (r, 0, 0)),
        out_shape=jax.ShapeDtypeStruct((B, N, _DIM), f32),
    )(x2, exp_sums, w1c, b1c, w2c, b2e)

    mi_loss = exp_sums / (H * W)
    return (x3, mi_loss)


# speedup vs baseline: 1.0766x; 1.0766x over previous
import jax
import jax.numpy as jnp
from jax.experimental import pallas as pl

_DIM = 768
_HEADS = 12
_HD = _DIM // _HEADS
_MLPH = 4 * _DIM
_E = 8
_HID = 192
_SCALE = _HD ** -0.5
_ABT = 2
_BT = 4
_CHUNKS = 4
_BF = jnp.bfloat16


def _gelu(x):
    return x * 0.5 * (1.0 + jax.lax.erf(x * (2.0 ** -0.5)))


def _ln(x, g, b):
    m = jnp.mean(x, axis=-1, keepdims=True)
    v = jnp.mean((x - m) ** 2, axis=-1, keepdims=True)
    return (x - m) * jax.lax.rsqrt(v + 1e-5) * g + b


def _dot(a, b):
    return jnp.dot(a, b, preferred_element_type=jnp.float32)


def _block_kernel(x_ref, wqkv_ref, wproj_ref, bproj_ref, g1_ref, b1_ref,
                  w1_ref, mb1_ref, w2_ref, mb2_ref, g2_ref, b2_ref,
                  gw_ref, eps_ref, o_ref, es_ref):
    x1s = []
    for i in range(_ABT):
        x = x_ref[i]
        xn = _ln(x, g1_ref[...], b1_ref[...]).astype(_BF)
        qkv = _dot(xn, wqkv_ref[...]).astype(_BF)
        outs = []
        for h in range(_HEADS):
            q = qkv[:, h * _HD:(h + 1) * _HD]
            k = qkv[:, _DIM + h * _HD:_DIM + (h + 1) * _HD]
            v = qkv[:, 2 * _DIM + h * _HD:2 * _DIM + (h + 1) * _HD]
            s = jax.lax.dot_general(q, k, (((1,), (1,)), ((), ())),
                                    preferred_element_type=jnp.float32)
            p = jax.nn.softmax(s * _SCALE, axis=-1)
            outs.append(_dot(p.astype(_BF), v))
        ao = jnp.concatenate(outs, axis=1)
        x1s.append(x + _dot(ao.astype(_BF), wproj_ref[...]) + bproj_ref[...])
    n = x_ref.shape[1]
    x1 = jnp.concatenate(x1s, axis=0)
    xn2 = _ln(x1, g2_ref[...], b2_ref[...])
    acc = x1 + mb2_ref[...]
    ck = _MLPH // _CHUNKS
    for c in range(_CHUNKS):
        h = _gelu(_dot(xn2, w1_ref[:, c * ck:(c + 1) * ck])
                  + mb1_ref[0, c * ck:(c + 1) * ck])
        acc = acc + _dot(h, w2_ref[c * ck:(c + 1) * ck, :])
    o_ref[...] = acc.reshape(_ABT, n, _DIM)
    t = _dot(acc, gw_ref[...])
    clean = t[:, :_E]
    std = jax.nn.softplus(t[:, _E:]) + 0.01
    logits = clean + eps_ref[...].reshape(_ABT * n, _E) * std
    es_ref[0] = logits.reshape(_ABT, n, _E).sum(axis=1)


def _expert_kernel(x_ref, es_ref, w1c_ref, b1c_ref, w2c_ref, b2e_ref, o_ref):
    s = es_ref[...]
    t1 = jnp.max(s, axis=1, keepdims=True)
    ii = jax.lax.broadcasted_iota(jnp.int32, s.shape, 1)
    first = jnp.min(jnp.where(s == t1, ii, _E), axis=1, keepdims=True)
    t2 = jnp.max(jnp.where(ii == first, -jnp.inf, s), axis=1, keepdims=True)
    m2 = jnp.max(t1)
    m1 = jnp.min(t2)
    row0 = jax.lax.broadcasted_iota(jnp.int32, t1.shape, 0) == 0
    t1_0 = jnp.sum(jnp.where(row0, t1, 0.0))
    t2_0 = jnp.sum(jnp.where(row0, t2, 0.0))
    a = (t1_0 - m1) / (m2 - m1)
    b = (t2_0 - m1) / (m2 - m1)
    ea, eb = jnp.exp(a), jnp.exp(b)
    g0 = ea / (ea + eb)
    g1 = eb / (ea + eb)

    bt, n, _ = x_ref.shape
    x2 = x_ref[...].reshape(bt * n, _DIM)
    h = _gelu(_dot(x2.astype(_BF), w1c_ref[...]) + b1c_ref[...])
    gi = jax.lax.broadcasted_iota(jnp.int32, (1, 2 * _HID), 1)
    hs = h * jnp.where(gi < _HID, g0, g1)
    out = (x2 + _dot(hs.astype(_BF), w2c_ref[...])
           + g0 * b2e_ref[0:1, :] + g1 * b2e_ref[1:2, :])
    o_ref[...] = out.reshape(bt, n, _DIM)


def kernel(x, H, W, pseudo_domain_label, norm1_g, norm1_b, norm2_g, norm2_b,
           wq, wkv, wproj, bproj, mlp_w1, mlp_b1, mlp_w2, mlp_b2,
           gate_w, exp_w1, exp_b1, exp_w2, exp_b2):
    B, N, _ = x.shape
    f32 = jnp.float32
    row2 = lambda a: a.reshape(1, -1)
    wqkv = jnp.concatenate([wq, wkv], axis=1).astype(_BF)

    with jax.ensure_compile_time_eval():
        eps = jax.random.normal(jax.random.key(42), (B, N, _E), f32)
    nstep = B // _ABT
    cw = lambda shp: pl.BlockSpec(shp, lambda r: tuple(0 for _ in shp))
    x2, es3 = pl.pallas_call(
        _block_kernel,
        grid=(nstep,),
        in_specs=[
            pl.BlockSpec((_ABT, N, _DIM), lambda r: (r, 0, 0)),
            cw((_DIM, 3 * _DIM)),
            cw((_DIM, _DIM)),
            cw((1, _DIM)),
            cw((1, _DIM)),
            cw((1, _DIM)),
            cw((_DIM, _MLPH)),
            cw((1, _MLPH)),
            cw((_MLPH, _DIM)),
            cw((1, _DIM)),
            cw((1, _DIM)),
            cw((1, _DIM)),
            cw((_DIM, 2 * _E)),
            pl.BlockSpec((_ABT, N, _E), lambda r: (r, 0, 0)),
        ],
        out_specs=[
            pl.BlockSpec((_ABT, N, _DIM), lambda r: (r, 0, 0)),
            pl.BlockSpec((1, _ABT, _E), lambda r: (r, 0, 0)),
        ],
        out_shape=[
            jax.ShapeDtypeStruct((B, N, _DIM), f32),
            jax.ShapeDtypeStruct((nstep, _ABT, _E), f32),
        ],
    )(x, wqkv, wproj.astype(_BF), row2(bproj), row2(norm1_g), row2(norm1_b),
      mlp_w1, row2(mlp_b1), mlp_w2, row2(mlp_b2),
      row2(norm2_g), row2(norm2_b), gate_w, eps)
    exp_sums = es3.reshape(B, _E)

    w1c = jnp.concatenate([exp_w1[0], exp_w1[1]], axis=1).astype(_BF)
    b1c = jnp.concatenate([exp_b1[0], exp_b1[1]]).reshape(1, 2 * _HID)
    w2c = jnp.concatenate([exp_w2[0], exp_w2[1]], axis=0).astype(_BF)
    b2e = exp_b2[:2]
    ntile = B // _BT
    x3 = pl.pallas_call(
        _expert_kernel,
        grid=(ntile,),
        in_specs=[
            pl.BlockSpec((_BT, N, _DIM), lambda r: (r, 0, 0)),
            pl.BlockSpec((B, _E), lambda r: (0, 0)),
            pl.BlockSpec((_DIM, 2 * _HID), lambda r: (0, 0)),
            pl.BlockSpec((1, 2 * _HID), lambda r: (0, 0)),
            pl.BlockSpec((2 * _HID, _DIM), lambda r: (0, 0)),
            pl.BlockSpec((2, _DIM), lambda r: (0, 0)),
        ],
        out_specs=pl.BlockSpec((_BT, N, _DIM), lambda r: (r, 0, 0)),
        out_shape=jax.ShapeDtypeStruct((B, N, _DIM), f32),
    )(x2, exp_sums, w1c, b1c, w2c, b2e)

    mi_loss = exp_sums / (H * W)
    return (x3, mi_loss)
